# baseline (speedup 1.0000x reference)
import jax
import jax.numpy as jnp
from jax import lax
from jax.experimental import pallas as pl
from jax.experimental.pallas import tpu as pltpu
from jax.experimental.pallas import tpu_sc as plsc

_VOCAB = 100000
_D = 64
_MAX_OFF = 128
_EPS = 1e-12
_B, _S = 4, 512

_TAB_ROWS = 1040
_KPAD = 264

_SC_CORES = 2
_SC_SUBCORES = 16
_NW = _SC_CORES * _SC_SUBCORES
_ROWS_PER_W = (_B * _S) // _NW
_PAIRS_PER_W = (_B * _S) // _NW
_QROWS = 128
_NQ = _S // _QROWS


def _layernorm_rows(x):
    u = jnp.mean(x, axis=-1, keepdims=True)
    s = jnp.mean((x - u) ** 2, axis=-1, keepdims=True)
    return (x - u) / jnp.sqrt(s + _EPS)


def _tab128_body(wrel_ref, out_ref):
    kk = lax.broadcasted_iota(jnp.int32, (_TAB_ROWS, _KPAD), 0)
    vv = lax.broadcasted_iota(jnp.int32, (_TAB_ROWS, _KPAD), 1)
    d = kk - (_S - 1)
    toepval = jnp.where(d >= 0, jnp.minimum(d, _MAX_OFF - 1),
                        jnp.maximum(d, -_MAX_OFF + 1) + 2 * _MAX_OFF)
    g = jnp.where(kk < 2 * _S - 1, toepval, 0)
    g = jnp.where(kk == 1024, _MAX_OFF, g)
    g = jnp.where(kk == 1025, 2 * _MAX_OFF, g)
    g = jnp.where(kk == 1026, 2 * _MAX_OFF + 1, g)
    onehot = (g == vv).astype(jnp.float32)
    ln_rel = _layernorm_rows(wrel_ref[...])
    res = jnp.dot(onehot, ln_rel, precision=lax.Precision.HIGHEST,
                  preferred_element_type=jnp.float32)
    out_ref[:, 0:_D] = res
    out_ref[:, _D:] = res


def _sc_embd2_body(tab_hbm, types_hbm, out_hbm,
                   t_v, idx_v, blk0, blk1, outb, sem0, sem1):
    wid = lax.axis_index("s") * _SC_CORES + lax.axis_index("c")
    pltpu.sync_copy(types_hbm, t_v)
    blks = (blk0, blk1)
    sems = (sem0, sem1)

    def pair_body(qp, carry):
        p = wid * _PAIRS_PER_W + qp
        b = p // _S
        i = p - b * _S
        fi = b * _S + i
        trow_i = fi // 128
        tcol_i = fi - trow_i * 128
        ti = plsc.load_gather(
            t_v, [jnp.full((16,), trow_i, jnp.int32),
                  jnp.full((16,), tcol_i, jnp.int32)])
        s = _S - 1 - i
        for jv in range(_S // 16):
            jvals = lax.iota(jnp.int32, 16) + jv * 16
            trow = b * 4 + jv // 8
            tt = t_v[trow, pl.ds((jv % 8) * 16, 16)]
            idxc = jnp.where(tt == ti, s + jvals, 1024)
            idxc = jnp.where((i == 0) & (jvals >= 1), 1025, idxc)
            idxc = jnp.where((i > 0) & (jvals == 0), 1026, idxc)
            idx_v[jv // 8, pl.ds((jv % 8) * 16, 16)] = idxc
        g = pltpu.async_copy(tab_hbm.at[idx_v.at[0]], blks[0], sems[0])
        for q in range(_NQ):
            g.wait()
            if q < _NQ - 1:
                g = pltpu.async_copy(tab_hbm.at[idx_v.at[q + 1]],
                                     blks[(q + 1) % 2], sems[(q + 1) % 2])
            qbase = q * _QROWS

            def row_copy(rr, _, q=q, qbase=qbase):
                for ch in range(_D // 16):
                    outb[qbase + rr, pl.ds(ch * 16, 16)] = (
                        blks[q % 2][rr, pl.ds(ch * 16, 16)])
                return 0

            lax.fori_loop(0, _QROWS, row_copy, 0)
        pltpu.sync_copy(outb, out_hbm.at[b, i])
        return carry

    lax.fori_loop(0, _PAIRS_PER_W, pair_body, 0)


def _make_sc_embd2():
    return pl.kernel(
        _sc_embd2_body,
        mesh=plsc.VectorSubcoreMesh(core_axis_name="c", subcore_axis_name="s"),
        out_type=jax.ShapeDtypeStruct((_B, _S, _S, _D), jnp.float32),
        scratch_types=[
            pltpu.VMEM((16, 128), jnp.int32),
            pltpu.VMEM((_NQ, _QROWS), jnp.int32),
            pltpu.VMEM((_QROWS, 2 * _D), jnp.float32),
            pltpu.VMEM((_QROWS, 2 * _D), jnp.float32),
            pltpu.VMEM((_S, _D), jnp.float32),
            pltpu.SemaphoreType.DMA,
            pltpu.SemaphoreType.DMA,
        ],
        compiler_params=pltpu.CompilerParams(needs_layout_passes=False),
    )


def _embd1_body(rows_ref, tcol_ref, wtype_ref, out_ref):
    mask = tcol_ref[0] == 0
    tw = jnp.where(mask, wtype_ref[0:1, :], wtype_ref[1:2, :])
    out_ref[0] = _layernorm_rows(rows_ref[0] + tw)


def _sc_word_gather_body(table_hbm, idx_hbm, out_hbm, idx_v, rows_v, sem):
    wid = lax.axis_index("s") * _SC_CORES + lax.axis_index("c")
    base = wid * _ROWS_PER_W
    pltpu.sync_copy(idx_hbm.at[pl.ds(base, _ROWS_PER_W)], idx_v)
    pltpu.async_copy(table_hbm.at[idx_v], rows_v, sem).wait()
    pltpu.sync_copy(rows_v, out_hbm.at[pl.ds(base, _ROWS_PER_W)])


def _make_sc_word_gather():
    return pl.kernel(
        _sc_word_gather_body,
        mesh=plsc.VectorSubcoreMesh(core_axis_name="c", subcore_axis_name="s"),
        out_type=jax.ShapeDtypeStruct((_B * _S, _D), jnp.float32),
        scratch_types=[
            pltpu.VMEM((_ROWS_PER_W,), jnp.int32),
            pltpu.VMEM((_ROWS_PER_W, _D), jnp.float32),
            pltpu.SemaphoreType.DMA,
        ],
        compiler_params=pltpu.CompilerParams(use_tc_tiling_on_sc=False),
    )


def kernel(tok_seq, tok_type_ids, W_word, W_type, W_rel):
    types_col = tok_type_ids[:, :, None]
    wrel_pad = jnp.zeros((_KPAD, _D), jnp.float32).at[: 2 * _MAX_OFF + 2].set(W_rel)

    word_rows = _make_sc_word_gather()(W_word, tok_seq.reshape(-1))

    tab128 = pl.pallas_call(
        _tab128_body,
        out_shape=jax.ShapeDtypeStruct((_TAB_ROWS, 2 * _D), jnp.float32),
    )(wrel_pad)

    types16 = tok_type_ids.reshape(16, 128)
    embd2 = _make_sc_embd2()(tab128, types16)

    embd1 = pl.pallas_call(
        _embd1_body,
        grid=(_B,),
        in_specs=[
            pl.BlockSpec((1, _S, _D), lambda b: (b, 0, 0)),
            pl.BlockSpec((1, _S, 1), lambda b: (b, 0, 0)),
            pl.BlockSpec((2, _D), lambda b: (0, 0)),
        ],
        out_specs=pl.BlockSpec((1, _S, _D), lambda b: (b, 0, 0)),
        out_shape=jax.ShapeDtypeStruct((_B, _S, _D), jnp.float32),
    )(word_rows.reshape(_B, _S, _D), types_col, W_type)

    return (embd1, embd2)


# speedup vs baseline: 33.7168x; 33.7168x over previous
import functools

import jax
import jax.numpy as jnp
from jax import lax
from jax.experimental import pallas as pl
from jax.experimental.pallas import tpu as pltpu
from jax.experimental.pallas import tpu_sc as plsc

_VOCAB = 100000
_D = 64
_MAX_OFF = 128
_EPS = 1e-12
_B, _S = 4, 512

_TAB_ROWS = 1040
_KPAD = 264
_BANKS = 8
_BI = 64


def _layernorm_rows(x):
    u = jnp.mean(x, axis=-1, keepdims=True)
    s = jnp.mean((x - u) ** 2, axis=-1, keepdims=True)
    return (x - u) / jnp.sqrt(s + _EPS)


def _embd2_body(wrel_ref, types_smem, tcol_ref, out_ref, tab_s):
    b = pl.program_id(0)
    ib = pl.program_id(1)

    @pl.when((b == 0) & (ib == 0))
    def _build_table():
        rr = lax.broadcasted_iota(jnp.int32, (_BANKS, _TAB_ROWS, _KPAD), 0)
        kk = lax.broadcasted_iota(jnp.int32, (_BANKS, _TAB_ROWS, _KPAD), 1)
        vv = lax.broadcasted_iota(jnp.int32, (_BANKS, _TAB_ROWS, _KPAD), 2)
        karg = kk + rr
        d = karg - (_S - 1)
        toepval = jnp.where(d >= 0, jnp.minimum(d, _MAX_OFF - 1),
                            jnp.maximum(d, -_MAX_OFF + 1) + 2 * _MAX_OFF)
        g = jnp.where(karg < 2 * _S - 1, toepval, 0)
        g = jnp.where(karg == 1024, _MAX_OFF, g)
        g = jnp.where(karg == 1025, 2 * _MAX_OFF, g)
        g = jnp.where(karg == 1026, 2 * _MAX_OFF + 1, g)
        onehot = (g == vv).astype(jnp.float32)
        onehot = onehot.reshape(_BANKS * _TAB_ROWS, _KPAD)
        ln_rel = _layernorm_rows(wrel_ref[...])
        tab_s[...] = jnp.dot(onehot, ln_rel,
                             precision=lax.Precision.HIGHEST,
                             preferred_element_type=jnp.float32)

    r128 = tab_s[1024:1025, :]
    for c in range(_BI):
        i = ib * _BI + c
        ti = types_smem[b, i]
        mask = tcol_ref[0] == ti
        s = _S - 1 - i
        r = lax.rem(s, 8)
        start = pl.multiple_of(r * _TAB_ROWS + (s - r), 8)
        win = tab_s[pl.ds(start, _S), :]
        base = jnp.where(mask, win, r128)

        if c == 0:
            @pl.when(ib == 0)
            def _first_row():
                jcol = lax.broadcasted_iota(jnp.int32, (_S, 1), 0)
                out_ref[0, 0] = jnp.where(jcol >= 1, tab_s[1025:1026, :], base)

            @pl.when(ib > 0)
            def _other_first():
                out_ref[0, 0] = base
                out_ref[0, 0, 0:1, :] = tab_s[1026:1027, :]
        else:
            out_ref[0, c] = base
            out_ref[0, c, 0:1, :] = tab_s[1026:1027, :]


def _embd1_body(rows_ref, tcol_ref, wtype_ref, out_ref):
    mask = tcol_ref[0] == 0
    tw = jnp.where(mask, wtype_ref[0:1, :], wtype_ref[1:2, :])
    out_ref[0] = _layernorm_rows(rows_ref[0] + tw)


_SC_CORES = 2
_SC_SUBCORES = 16
_NW = _SC_CORES * _SC_SUBCORES
_ROWS_PER_W = (_B * _S) // _NW


def _sc_word_gather_body(table_hbm, idx_hbm, out_hbm, idx_v, rows_v, sem):
    wid = lax.axis_index("s") * _SC_CORES + lax.axis_index("c")
    base = wid * _ROWS_PER_W
    pltpu.sync_copy(idx_hbm.at[pl.ds(base, _ROWS_PER_W)], idx_v)
    pltpu.async_copy(table_hbm.at[idx_v], rows_v, sem).wait()
    pltpu.sync_copy(rows_v, out_hbm.at[pl.ds(base, _ROWS_PER_W)])


def _make_sc_word_gather():
    return pl.kernel(
        _sc_word_gather_body,
        mesh=plsc.VectorSubcoreMesh(core_axis_name="c", subcore_axis_name="s"),
        out_type=jax.ShapeDtypeStruct((_B * _S, _D), jnp.float32),
        scratch_types=[
            pltpu.VMEM((_ROWS_PER_W,), jnp.int32),
            pltpu.VMEM((_ROWS_PER_W, _D), jnp.float32),
            pltpu.SemaphoreType.DMA,
        ],
        compiler_params=pltpu.CompilerParams(use_tc_tiling_on_sc=False),
    )


def kernel(tok_seq, tok_type_ids, W_word, W_type, W_rel):
    types_col = tok_type_ids[:, :, None]
    wrel_pad = jnp.zeros((_KPAD, _D), jnp.float32).at[: 2 * _MAX_OFF + 2].set(W_rel)

    word_rows = _make_sc_word_gather()(W_word, tok_seq.reshape(-1))

    embd2 = pl.pallas_call(
        _embd2_body,
        grid=(_B, _S // _BI),
        in_specs=[
            pl.BlockSpec((_KPAD, _D), lambda b, i: (0, 0)),
            pl.BlockSpec(memory_space=pltpu.SMEM),
            pl.BlockSpec((1, _S, 1), lambda b, i: (b, 0, 0)),
        ],
        out_specs=pl.BlockSpec((1, _BI, _S, _D), lambda b, i: (b, i, 0, 0)),
        out_shape=jax.ShapeDtypeStruct((_B, _S, _S, _D), jnp.float32),
        scratch_shapes=[pltpu.VMEM((_BANKS * _TAB_ROWS, _D), jnp.float32)],
        compiler_params=pltpu.CompilerParams(
            dimension_semantics=("arbitrary", "arbitrary"),
            vmem_limit_bytes=60000 * 1024),
    )(wrel_pad, tok_type_ids, types_col)

    embd1 = pl.pallas_call(
        _embd1_body,
        grid=(_B,),
        in_specs=[
            pl.BlockSpec((1, _S, _D), lambda b: (b, 0, 0)),
            pl.BlockSpec((1, _S, 1), lambda b: (b, 0, 0)),
            pl.BlockSpec((2, _D), lambda b: (0, 0)),
        ],
        out_specs=pl.BlockSpec((1, _S, _D), lambda b: (b, 0, 0)),
        out_shape=jax.ShapeDtypeStruct((_B, _S, _D), jnp.float32),
    )(word_rows.reshape(_B, _S, _D), types_col, W_type)

    return (embd1, embd2)
